# Initial kernel scaffold; baseline (speedup 1.0000x reference)
import functools

import jax
import jax.numpy as jnp
from jax import lax
from jax.experimental import pallas as pl
from jax.experimental.pallas import tpu as pltpu
from jax.experimental.pallas import tpu_sc as plsc

N = 10000
E = 320000
D = 128
D_OUT = 64

NC = 2
NS = 16
NW = NC * NS

EPW = E // NW
C = 80
NCHUNK = EPW // C

RPT = N // NS
NP = 10240
RPT_D = NP // NS


def _sc_deg_body(dst_hbm, ones_hbm, zeros_hbm, out_hbm, didx, ones_v, acc):
    c = lax.axis_index("c")
    s = lax.axis_index("s")
    wid = c * NS + s
    ebase = wid * EPW

    pltpu.sync_copy(zeros_hbm.at[pl.ds(s * RPT_D, RPT_D)],
                    acc.at[pl.ds(s * RPT_D, RPT_D)])
    pltpu.sync_copy(ones_hbm, ones_v)
    plsc.subcore_barrier()

    def step(j, carry):
        pltpu.sync_copy(dst_hbm.at[pl.ds(ebase + j * C, C)], didx)
        pltpu.sync_copy(ones_v, acc.at[didx], add=True)
        return carry

    lax.fori_loop(0, NCHUNK, step, 0)
    plsc.subcore_barrier()

    pltpu.sync_copy(acc.at[pl.ds(s * RPT_D, RPT_D)],
                    out_hbm.at[c, pl.ds(s * RPT_D, RPT_D)])


@jax.jit
def _sc_deg(dst, ones_c, zeros_np):
    return pl.kernel(
        _sc_deg_body,
        out_type=jax.ShapeDtypeStruct((NC, NP, 1), jnp.float32),
        mesh=plsc.VectorSubcoreMesh(core_axis_name="c", subcore_axis_name="s"),
        scratch_types=[
            pltpu.VMEM((C,), jnp.int32),
            pltpu.VMEM((C, 1), jnp.float32),
            pltpu.VMEM_SHARED((NP, 1), jnp.float32),
        ],
    )(dst, ones_c, zeros_np)


def _sc_agg_body(g_hbm, src_hbm, dst_hbm, zeros_hbm, out_hbm,
                 sidx, didx, rows, acc, gsem):
    c = lax.axis_index("c")
    s = lax.axis_index("s")
    wid = c * NS + s
    ebase = wid * EPW

    pltpu.sync_copy(zeros_hbm.at[pl.ds(s * RPT, RPT)],
                    acc.at[pl.ds(s * RPT, RPT)])
    plsc.subcore_barrier()

    def step(j, carry):
        base = ebase + j * C
        pltpu.sync_copy(src_hbm.at[pl.ds(base, C)], sidx)
        pltpu.sync_copy(dst_hbm.at[pl.ds(base, C)], didx)
        pltpu.async_copy(g_hbm.at[sidx], rows, gsem).wait()
        pltpu.sync_copy(rows, acc.at[didx], add=True)
        return carry

    lax.fori_loop(0, NCHUNK, step, 0)
    plsc.subcore_barrier()

    pltpu.sync_copy(acc.at[pl.ds(s * RPT, RPT)],
                    out_hbm.at[c, pl.ds(s * RPT, RPT)])


@jax.jit
def _sc_agg(g, src, dst, zeros_nd):
    return pl.kernel(
        _sc_agg_body,
        out_type=jax.ShapeDtypeStruct((NC, N, D), jnp.float32),
        mesh=plsc.VectorSubcoreMesh(core_axis_name="c", subcore_axis_name="s"),
        scratch_types=[
            pltpu.VMEM((C,), jnp.int32),
            pltpu.VMEM((C,), jnp.int32),
            pltpu.VMEM((C, D), jnp.float32),
            pltpu.VMEM_SHARED((N, D), jnp.float32),
            pltpu.SemaphoreType.DMA,
        ],
    )(g, src, dst, zeros_nd)


def _tc1_body(degp_ref, x_ref, w_ref, dinv_ref, g_ref):
    deg = degp_ref[0] + degp_ref[1] + 1.0
    dinv = lax.rsqrt(deg)[:N]
    dinv_ref[...] = dinv
    h = jnp.dot(x_ref[...], w_ref[...], preferred_element_type=jnp.float32)
    g_ref[...] = dinv * h


@jax.jit
def _tc1(degp, x, w1):
    return pl.pallas_call(
        _tc1_body,
        out_shape=(
            jax.ShapeDtypeStruct((N, 1), jnp.float32),
            jax.ShapeDtypeStruct((N, D), jnp.float32),
        ),
    )(degp, x, w1)


def _tc2_body(p_ref, g_ref, dinv_ref, b_ref, w_ref, g2_ref):
    ssum = p_ref[0] + p_ref[1] + g_ref[...]
    h = jnp.maximum(dinv_ref[...] * ssum + b_ref[...], 0.0)
    hw = jnp.dot(h, w_ref[...], preferred_element_type=jnp.float32)
    g2_ref[...] = dinv_ref[...] * hw


@jax.jit
def _tc2(p, g, dinv, b, w2):
    return pl.pallas_call(
        _tc2_body,
        out_shape=jax.ShapeDtypeStruct((N, D), jnp.float32),
    )(p, g, dinv, b, w2)


def _tc3_body(p_ref, g_ref, dinv_ref, b_ref, wr1_ref, br1_ref, wr2_ref,
              br2_ref, o_ref):
    ssum = p_ref[0] + p_ref[1] + g_ref[...]
    h = jnp.maximum(dinv_ref[...] * ssum + b_ref[...], 0.0)
    t = jnp.maximum(
        jnp.dot(h, wr1_ref[...], preferred_element_type=jnp.float32)
        + br1_ref[...], 0.0)
    o_ref[...] = (
        jnp.dot(t, wr2_ref[...], preferred_element_type=jnp.float32)
        + br2_ref[...])


@jax.jit
def _tc3(p, g, dinv, b, wr1, br1, wr2, br2):
    return pl.pallas_call(
        _tc3_body,
        out_shape=jax.ShapeDtypeStruct((N, D_OUT), jnp.float32),
    )(p, g, dinv, b, wr1, br1, wr2, br2)


@jax.jit
def kernel(x, edge_index, W1, b1, W2, b2, Wr1, br1, Wr2, br2):
    src = edge_index[0]
    dst = edge_index[1]
    ones_c = jnp.ones((C, 1), jnp.float32)
    zeros_np = jnp.zeros((NP, 1), jnp.float32)
    zeros_nd = jnp.zeros((N, D), jnp.float32)

    degp = _sc_deg(dst, ones_c, zeros_np)
    dinv, g1 = _tc1(degp, x, W1)
    p1 = _sc_agg(g1, src, dst, zeros_nd)
    g2 = _tc2(p1, g1, dinv, b1.reshape(1, D), W2)
    p2 = _sc_agg(g2, src, dst, zeros_nd)
    out = _tc3(p2, g2, dinv, b2.reshape(1, D), Wr1, br1.reshape(1, -1),
               Wr2, br2.reshape(1, -1))
    return out


# speedup vs baseline: 13.0632x; 13.0632x over previous
import jax
import jax.numpy as jnp
from jax import lax
from jax.experimental import pallas as pl
from jax.experimental.pallas import tpu as pltpu
from jax.experimental.pallas import tpu_sc as plsc

N = 10000
E = 320000
D = 128
D_OUT = 64

NC = 2
NS = 16
NW = NC * NS

EPW = E // NW
C = 80
NCHUNK = EPW // C

NP = 10240
RPT = NP // NS
RPT_D = RPT


HR = NP // 128


def _sc_deg_body(dst_hbm, out_hbm, didx, rowidx, zrows, hist, acc):
    c = lax.axis_index("c")
    s = lax.axis_index("s")
    wid = c * NS + s
    ebase = wid * EPW
    zero16 = jnp.zeros((16,), jnp.float32)
    one16 = jnp.ones((16,), jnp.float32)

    def zhist(i, carry):
        hist[i // 8, pl.ds((i % 8) * 16, 16)] = zero16
        return carry

    lax.fori_loop(0, HR * 8, zhist, 0)

    for j in range(HR // 16):
        rowidx[pl.ds(j * 16, 16)] = lax.iota(jnp.int32, 16) + j * 16

    @pl.when(s < 5)
    def _():
        def zr(i, carry):
            zrows[i // 8, pl.ds((i % 8) * 16, 16)] = zero16
            return carry
        lax.fori_loop(0, 16 * 8, zr, 0)
        pltpu.sync_copy(zrows, acc.at[pl.ds(s * 16, 16)])
    plsc.subcore_barrier()

    def step(j, carry):
        pltpu.sync_copy(dst_hbm.at[pl.ds(ebase + j * C, C)], didx)
        for k in range(C // 16):
            idx = didx[pl.ds(k * 16, 16)]
            r = jax.lax.shift_right_logical(idx, 7)
            l = jax.lax.bitwise_and(idx, 127)
            plsc.addupdate_scatter(hist, [r, l], one16)
        return carry

    lax.fori_loop(0, NCHUNK, step, 0)

    pltpu.sync_copy(hist, acc.at[rowidx], add=True)
    plsc.subcore_barrier()

    @pl.when(s < 5)
    def _():
        pltpu.sync_copy(acc.at[pl.ds(s * 16, 16)],
                        out_hbm.at[c, pl.ds(s * 16, 16)])


@jax.jit
def _sc_deg(dst):
    return pl.kernel(
        _sc_deg_body,
        out_type=jax.ShapeDtypeStruct((NC, HR, 128), jnp.float32),
        mesh=plsc.VectorSubcoreMesh(core_axis_name="c", subcore_axis_name="s"),
        scratch_types=[
            pltpu.VMEM((C,), jnp.int32),
            pltpu.VMEM((HR,), jnp.int32),
            pltpu.VMEM((16, 128), jnp.float32),
            pltpu.VMEM((HR, 128), jnp.float32),
            pltpu.VMEM_SHARED((HR, 128), jnp.float32),
        ],
        compiler_params=pltpu.CompilerParams(needs_layout_passes=False),
    )(dst)


def _sc_agg_body(g_hbm, src_hbm, dst_hbm, zeros_hbm, out_hbm,
                 sidx, didx, rows, acc, gsem):
    c = lax.axis_index("c")
    s = lax.axis_index("s")
    wid = c * NS + s
    ebase = wid * EPW

    pltpu.sync_copy(zeros_hbm.at[pl.ds(s * RPT, RPT)],
                    acc.at[pl.ds(s * RPT, RPT)])
    plsc.subcore_barrier()

    def step(j, carry):
        base = ebase + j * C
        pltpu.sync_copy(src_hbm.at[pl.ds(base, C)], sidx)
        pltpu.sync_copy(dst_hbm.at[pl.ds(base, C)], didx)
        pltpu.async_copy(g_hbm.at[sidx], rows, gsem).wait()
        pltpu.sync_copy(rows, acc.at[didx], add=True)
        return carry

    lax.fori_loop(0, NCHUNK, step, 0)
    plsc.subcore_barrier()

    pltpu.sync_copy(acc.at[pl.ds(s * RPT, RPT)],
                    out_hbm.at[c, pl.ds(s * RPT, RPT)])


@jax.jit
def _sc_agg(g, src, dst, zeros_nd):
    return pl.kernel(
        _sc_agg_body,
        out_type=jax.ShapeDtypeStruct((NC, NP, D), jnp.float32),
        mesh=plsc.VectorSubcoreMesh(core_axis_name="c", subcore_axis_name="s"),
        scratch_types=[
            pltpu.VMEM((C,), jnp.int32),
            pltpu.VMEM((C,), jnp.int32),
            pltpu.VMEM((C, D), jnp.float32),
            pltpu.VMEM_SHARED((NP, D), jnp.float32),
            pltpu.SemaphoreType.DMA,
        ],
    )(g, src, dst, zeros_nd)


def _tc1_body(degp_ref, x_ref, w_ref, dinv_ref, g_ref):
    deg = degp_ref[0] + degp_ref[1] + 1.0
    dinv = lax.rsqrt(deg)[:N]
    dinv_ref[...] = dinv
    h = jnp.dot(x_ref[...], w_ref[...], preferred_element_type=jnp.float32)
    g_ref[...] = dinv * h


@jax.jit
def _tc1(degp, x, w1):
    return pl.pallas_call(
        _tc1_body,
        out_shape=(
            jax.ShapeDtypeStruct((N, 1), jnp.float32),
            jax.ShapeDtypeStruct((N, D), jnp.float32),
        ),
    )(degp, x, w1)


def _tc2_body(p_ref, g_ref, dinv_ref, b_ref, w_ref, g2_ref):
    ssum = (p_ref[0] + p_ref[1])[:N] + g_ref[...]
    h = jnp.maximum(dinv_ref[...] * ssum + b_ref[...], 0.0)
    hw = jnp.dot(h, w_ref[...], preferred_element_type=jnp.float32)
    g2_ref[...] = dinv_ref[...] * hw


@jax.jit
def _tc2(p, g, dinv, b, w2):
    return pl.pallas_call(
        _tc2_body,
        out_shape=jax.ShapeDtypeStruct((N, D), jnp.float32),
    )(p, g, dinv, b, w2)


def _tc3_body(p_ref, g_ref, dinv_ref, b_ref, wr1_ref, br1_ref, wr2_ref,
              br2_ref, o_ref):
    ssum = (p_ref[0] + p_ref[1])[:N] + g_ref[...]
    h = jnp.maximum(dinv_ref[...] * ssum + b_ref[...], 0.0)
    t = jnp.maximum(
        jnp.dot(h, wr1_ref[...], preferred_element_type=jnp.float32)
        + br1_ref[...], 0.0)
    o_ref[...] = (
        jnp.dot(t, wr2_ref[...], preferred_element_type=jnp.float32)
        + br2_ref[...])


@jax.jit
def _tc3(p, g, dinv, b, wr1, br1, wr2, br2):
    return pl.pallas_call(
        _tc3_body,
        out_shape=jax.ShapeDtypeStruct((N, D_OUT), jnp.float32),
    )(p, g, dinv, b, wr1, br1, wr2, br2)


@jax.jit
def kernel(x, edge_index, W1, b1, W2, b2, Wr1, br1, Wr2, br2):
    src = edge_index[0]
    dst = edge_index[1]
    zeros_nd = jnp.zeros((NP, D), jnp.float32)

    degp = _sc_deg(dst).reshape(NC, NP, 1)
    dinv, g1 = _tc1(degp, x, W1)
    p1 = _sc_agg(g1, src, dst, zeros_nd)
    g2 = _tc2(p1, g1, dinv, b1.reshape(1, D), W2)
    p2 = _sc_agg(g2, src, dst, zeros_nd)
    out = _tc3(p2, g2, dinv, b2.reshape(1, D), Wr1, br1.reshape(1, -1),
               Wr2, br2.reshape(1, -1))
    return out


# speedup vs baseline: 25.9613x; 1.9874x over previous
import jax
import jax.numpy as jnp
from jax import lax
from jax.experimental import pallas as pl
from jax.experimental.pallas import tpu as pltpu
from jax.experimental.pallas import tpu_sc as plsc

N = 10000
E = 320000
D = 128
D_OUT = 64

NC = 2
NS = 16
NW = NC * NS

EPW = E // NW
C = 80
NCHUNK = EPW // C

NP = 10240
RPT = NP // NS
RPT_D = RPT


HR = NP // 128


def _sc_deg_body(dst_hbm, out_hbm, didx, rowidx, zrows, hist, acc):
    c = lax.axis_index("c")
    s = lax.axis_index("s")
    wid = c * NS + s
    ebase = wid * EPW
    zero16 = jnp.zeros((16,), jnp.float32)
    one16 = jnp.ones((16,), jnp.float32)

    def zhist(i, carry):
        hist[i // 8, pl.ds((i % 8) * 16, 16)] = zero16
        return carry

    lax.fori_loop(0, HR * 8, zhist, 0)

    for j in range(HR // 16):
        rowidx[pl.ds(j * 16, 16)] = lax.iota(jnp.int32, 16) + j * 16

    @pl.when(s < 5)
    def _():
        def zr(i, carry):
            zrows[i // 8, pl.ds((i % 8) * 16, 16)] = zero16
            return carry
        lax.fori_loop(0, 16 * 8, zr, 0)
        pltpu.sync_copy(zrows, acc.at[pl.ds(s * 16, 16)])
    plsc.subcore_barrier()

    def step(j, carry):
        pltpu.sync_copy(dst_hbm.at[pl.ds(ebase + j * C, C)], didx)
        for k in range(C // 16):
            idx = didx[pl.ds(k * 16, 16)]
            r = jax.lax.shift_right_logical(idx, 7)
            l = jax.lax.bitwise_and(idx, 127)
            plsc.addupdate_scatter(hist, [r, l], one16)
        return carry

    lax.fori_loop(0, NCHUNK, step, 0)

    pltpu.sync_copy(hist, acc.at[rowidx], add=True)
    plsc.subcore_barrier()

    @pl.when(s < 5)
    def _():
        pltpu.sync_copy(acc.at[pl.ds(s * 16, 16)],
                        out_hbm.at[c, pl.ds(s * 16, 16)])


@jax.jit
def _sc_deg(dst):
    return pl.kernel(
        _sc_deg_body,
        out_type=jax.ShapeDtypeStruct((NC, HR, 128), jnp.float32),
        mesh=plsc.VectorSubcoreMesh(core_axis_name="c", subcore_axis_name="s"),
        scratch_types=[
            pltpu.VMEM((C,), jnp.int32),
            pltpu.VMEM((HR,), jnp.int32),
            pltpu.VMEM((16, 128), jnp.float32),
            pltpu.VMEM((HR, 128), jnp.float32),
            pltpu.VMEM_SHARED((HR, 128), jnp.float32),
        ],
        compiler_params=pltpu.CompilerParams(needs_layout_passes=False),
    )(dst)


def _sc_agg_body(g_hbm, src_hbm, dst_hbm, zeros_hbm, out_hbm,
                 sidx_all, didx_all, didx0, didx1, rows0, rows1,
                 acc, gsem0, gsem1):
    c = lax.axis_index("c")
    s = lax.axis_index("s")
    wid = c * NS + s
    ebase = wid * EPW

    pltpu.sync_copy(zeros_hbm.at[pl.ds(s * RPT, RPT)],
                    acc.at[pl.ds(s * RPT, RPT)])
    pltpu.sync_copy(src_hbm.at[pl.ds(ebase, EPW)], sidx_all)
    pltpu.sync_copy(dst_hbm.at[pl.ds(ebase, EPW)], didx_all)
    plsc.subcore_barrier()

    def gather_start(j, rows, gsem):
        return pltpu.async_copy(g_hbm.at[sidx_all.at[pl.ds(j * C, C)]],
                                rows, gsem)

    def didx_copy(j, didx):
        for k in range(C // 16):
            didx[pl.ds(k * 16, 16)] = didx_all[pl.ds(j * C + k * 16, 16)]

    didx_copy(0, didx0)
    gather_start(0, rows0, gsem0)

    def pair(i, carry):
        j0 = 2 * i
        didx_copy(j0 + 1, didx1)
        gather_start(j0 + 1, rows1, gsem1)
        pltpu.make_async_copy(g_hbm.at[sidx_all.at[pl.ds(j0 * C, C)]],
                              rows0, gsem0).wait()
        pltpu.sync_copy(rows0, acc.at[didx0], add=True)
        didx_copy(j0 + 2, didx0)
        gather_start(j0 + 2, rows0, gsem0)
        pltpu.make_async_copy(g_hbm.at[sidx_all.at[pl.ds((j0 + 1) * C, C)]],
                              rows1, gsem1).wait()
        pltpu.sync_copy(rows1, acc.at[didx1], add=True)
        return carry

    lax.fori_loop(0, (NCHUNK - 1) // 2, pair, 0)

    last = NCHUNK - 1
    pltpu.make_async_copy(g_hbm.at[sidx_all.at[pl.ds(last * C, C)]],
                          rows0, gsem0).wait()
    pltpu.sync_copy(rows0, acc.at[didx0], add=True)
    plsc.subcore_barrier()

    pltpu.sync_copy(acc.at[pl.ds(s * RPT, RPT)],
                    out_hbm.at[c, pl.ds(s * RPT, RPT)])


@jax.jit
def _sc_agg(g, src, dst, zeros_nd):
    return pl.kernel(
        _sc_agg_body,
        out_type=jax.ShapeDtypeStruct((NC, NP, D), jnp.float32),
        mesh=plsc.VectorSubcoreMesh(core_axis_name="c", subcore_axis_name="s"),
        scratch_types=[
            pltpu.VMEM((EPW,), jnp.int32),
            pltpu.VMEM((EPW,), jnp.int32),
            pltpu.VMEM((C,), jnp.int32),
            pltpu.VMEM((C,), jnp.int32),
            pltpu.VMEM((C, D), jnp.float32),
            pltpu.VMEM((C, D), jnp.float32),
            pltpu.VMEM_SHARED((NP, D), jnp.float32),
            pltpu.SemaphoreType.DMA,
            pltpu.SemaphoreType.DMA,
        ],
    )(g, src, dst, zeros_nd)


def _tc1_body(degp_ref, x_ref, w_ref, dinv_ref, g_ref):
    deg = degp_ref[0] + degp_ref[1] + 1.0
    dinv = lax.rsqrt(deg)[:N]
    dinv_ref[...] = dinv
    h = jnp.dot(x_ref[...], w_ref[...], preferred_element_type=jnp.float32)
    g_ref[...] = dinv * h


@jax.jit
def _tc1(degp, x, w1):
    return pl.pallas_call(
        _tc1_body,
        out_shape=(
            jax.ShapeDtypeStruct((N, 1), jnp.float32),
            jax.ShapeDtypeStruct((N, D), jnp.float32),
        ),
    )(degp, x, w1)


def _tc2_body(p_ref, g_ref, dinv_ref, b_ref, w_ref, g2_ref):
    ssum = (p_ref[0] + p_ref[1])[:N] + g_ref[...]
    h = jnp.maximum(dinv_ref[...] * ssum + b_ref[...], 0.0)
    hw = jnp.dot(h, w_ref[...], preferred_element_type=jnp.float32)
    g2_ref[...] = dinv_ref[...] * hw


@jax.jit
def _tc2(p, g, dinv, b, w2):
    return pl.pallas_call(
        _tc2_body,
        out_shape=jax.ShapeDtypeStruct((N, D), jnp.float32),
    )(p, g, dinv, b, w2)


def _tc3_body(p_ref, g_ref, dinv_ref, b_ref, wr1_ref, br1_ref, wr2_ref,
              br2_ref, o_ref):
    ssum = (p_ref[0] + p_ref[1])[:N] + g_ref[...]
    h = jnp.maximum(dinv_ref[...] * ssum + b_ref[...], 0.0)
    t = jnp.maximum(
        jnp.dot(h, wr1_ref[...], preferred_element_type=jnp.float32)
        + br1_ref[...], 0.0)
    o_ref[...] = (
        jnp.dot(t, wr2_ref[...], preferred_element_type=jnp.float32)
        + br2_ref[...])


@jax.jit
def _tc3(p, g, dinv, b, wr1, br1, wr2, br2):
    return pl.pallas_call(
        _tc3_body,
        out_shape=jax.ShapeDtypeStruct((N, D_OUT), jnp.float32),
    )(p, g, dinv, b, wr1, br1, wr2, br2)


@jax.jit
def kernel(x, edge_index, W1, b1, W2, b2, Wr1, br1, Wr2, br2):
    src = edge_index[0]
    dst = edge_index[1]
    zeros_nd = jnp.zeros((NP, D), jnp.float32)

    degp = _sc_deg(dst).reshape(NC, NP, 1)
    dinv, g1 = _tc1(degp, x, W1)
    p1 = _sc_agg(g1, src, dst, zeros_nd)
    g2 = _tc2(p1, g1, dinv, b1.reshape(1, D), W2)
    p2 = _sc_agg(g2, src, dst, zeros_nd)
    out = _tc3(p2, g2, dinv, b2.reshape(1, D), Wr1, br1.reshape(1, -1),
               Wr2, br2.reshape(1, -1))
    return out


# speedup vs baseline: 32.5491x; 1.2538x over previous
import jax
import jax.numpy as jnp
from jax import lax
from jax.experimental import pallas as pl
from jax.experimental.pallas import tpu as pltpu
from jax.experimental.pallas import tpu_sc as plsc

N = 10000
E = 320000
D = 128
D_OUT = 64

NC = 2
NS = 16
NW = NC * NS

EPW = E // NW
C = 128
NB = 78
CT = EPW - NB * C

NP = 10240
RPT = NP // NS
RPT_D = RPT


HR = NP // 128


def _sc_deg_body(dst_hbm, out_hbm, didx_all, rowidx, zrows, hist, acc):
    c = lax.axis_index("c")
    s = lax.axis_index("s")
    wid = c * NS + s
    ebase = wid * EPW
    zero16 = jnp.zeros((16,), jnp.float32)
    one16 = jnp.ones((16,), jnp.float32)

    def zhist(i, carry):
        hist[i // 8, pl.ds((i % 8) * 16, 16)] = zero16
        return carry

    lax.fori_loop(0, HR * 8, zhist, 0)

    for j in range(HR // 16):
        rowidx[pl.ds(j * 16, 16)] = lax.iota(jnp.int32, 16) + j * 16

    @pl.when(s < 5)
    def _():
        def zr(i, carry):
            zrows[i // 8, pl.ds((i % 8) * 16, 16)] = zero16
            return carry
        lax.fori_loop(0, 16 * 8, zr, 0)
        pltpu.sync_copy(zrows, acc.at[pl.ds(s * 16, 16)])
    plsc.subcore_barrier()

    pltpu.sync_copy(dst_hbm.at[pl.ds(ebase, EPW)], didx_all)

    def step(j, carry):
        for k in range(4):
            idx = didx_all[pl.ds(j * 64 + k * 16, 16)]
            r = jax.lax.shift_right_logical(idx, 7)
            l = jax.lax.bitwise_and(idx, 127)
            plsc.addupdate_scatter(hist, [r, l], one16)
        return carry

    lax.fori_loop(0, EPW // 64, step, 0)

    pltpu.sync_copy(hist, acc.at[rowidx], add=True)
    plsc.subcore_barrier()

    @pl.when(s < 5)
    def _():
        pltpu.sync_copy(acc.at[pl.ds(s * 16, 16)],
                        out_hbm.at[c, pl.ds(s * 16, 16)])


@jax.jit
def _sc_deg(dst):
    return pl.kernel(
        _sc_deg_body,
        out_type=jax.ShapeDtypeStruct((NC, HR, 128), jnp.float32),
        mesh=plsc.VectorSubcoreMesh(core_axis_name="c", subcore_axis_name="s"),
        scratch_types=[
            pltpu.VMEM((EPW,), jnp.int32),
            pltpu.VMEM((HR,), jnp.int32),
            pltpu.VMEM((16, 128), jnp.float32),
            pltpu.VMEM((HR, 128), jnp.float32),
            pltpu.VMEM_SHARED((HR, 128), jnp.float32),
        ],
        compiler_params=pltpu.CompilerParams(needs_layout_passes=False),
    )(dst)


def _sc_agg_body(g_hbm, src_hbm, dst_hbm, zeros_hbm, out_hbm,
                 sidx_all, didx0, didx1, tidx, rows0, rows1, rowst,
                 acc, gsem0, gsem1, isem0, isem1):
    c = lax.axis_index("c")
    s = lax.axis_index("s")
    wid = c * NS + s
    ebase = wid * EPW

    pltpu.sync_copy(zeros_hbm.at[pl.ds(s * RPT, RPT)],
                    acc.at[pl.ds(s * RPT, RPT)])
    pltpu.sync_copy(src_hbm.at[pl.ds(ebase, EPW)], sidx_all)
    plsc.subcore_barrier()

    def gather_start(j, rows, gsem):
        pltpu.async_copy(g_hbm.at[sidx_all.at[pl.ds(j * C, C)]], rows, gsem)

    def gather_wait(j, rows, gsem):
        pltpu.make_async_copy(g_hbm.at[sidx_all.at[pl.ds(j * C, C)]],
                              rows, gsem).wait()

    def didx_start(j, didx, isem):
        pltpu.async_copy(dst_hbm.at[pl.ds(ebase + j * C, C)], didx, isem)

    def didx_wait(j, didx, isem):
        pltpu.make_async_copy(dst_hbm.at[pl.ds(ebase + j * C, C)],
                              didx, isem).wait()

    didx_start(0, didx0, isem0)
    gather_start(0, rows0, gsem0)

    def pair(i, carry):
        j0 = 2 * i
        didx_start(j0 + 1, didx1, isem1)
        gather_start(j0 + 1, rows1, gsem1)
        gather_wait(j0, rows0, gsem0)
        didx_wait(j0, didx0, isem0)
        pltpu.sync_copy(rows0, acc.at[didx0], add=True)
        didx_start(j0 + 2, didx0, isem0)
        gather_start(j0 + 2, rows0, gsem0)
        gather_wait(j0 + 1, rows1, gsem1)
        didx_wait(j0 + 1, didx1, isem1)
        pltpu.sync_copy(rows1, acc.at[didx1], add=True)
        return carry

    lax.fori_loop(0, NB // 2 - 1, pair, 0)

    didx_start(NB - 1, didx1, isem1)
    gather_start(NB - 1, rows1, gsem1)
    gather_wait(NB - 2, rows0, gsem0)
    didx_wait(NB - 2, didx0, isem0)
    pltpu.sync_copy(rows0, acc.at[didx0], add=True)
    pltpu.sync_copy(dst_hbm.at[pl.ds(ebase + NB * C, CT)], tidx)
    pltpu.async_copy(g_hbm.at[sidx_all.at[pl.ds(NB * C, CT)]], rowst, gsem0)
    gather_wait(NB - 1, rows1, gsem1)
    didx_wait(NB - 1, didx1, isem1)
    pltpu.sync_copy(rows1, acc.at[didx1], add=True)
    pltpu.make_async_copy(g_hbm.at[sidx_all.at[pl.ds(NB * C, CT)]],
                          rowst, gsem0).wait()
    pltpu.sync_copy(rowst, acc.at[tidx], add=True)
    plsc.subcore_barrier()

    pltpu.sync_copy(acc.at[pl.ds(s * RPT, RPT)],
                    out_hbm.at[c, pl.ds(s * RPT, RPT)])


@jax.jit
def _sc_agg(g, src, dst, zeros_nd):
    return pl.kernel(
        _sc_agg_body,
        out_type=jax.ShapeDtypeStruct((NC, NP, D), jnp.float32),
        mesh=plsc.VectorSubcoreMesh(core_axis_name="c", subcore_axis_name="s"),
        scratch_types=[
            pltpu.VMEM((EPW,), jnp.int32),
            pltpu.VMEM((C,), jnp.int32),
            pltpu.VMEM((C,), jnp.int32),
            pltpu.VMEM((CT,), jnp.int32),
            pltpu.VMEM((C, D), jnp.float32),
            pltpu.VMEM((C, D), jnp.float32),
            pltpu.VMEM((CT, D), jnp.float32),
            pltpu.VMEM_SHARED((NP, D), jnp.float32),
            pltpu.SemaphoreType.DMA,
            pltpu.SemaphoreType.DMA,
            pltpu.SemaphoreType.DMA,
            pltpu.SemaphoreType.DMA,
        ],
    )(g, src, dst, zeros_nd)


def _tc1_body(degp_ref, x_ref, w_ref, dinv_ref, g_ref):
    deg = degp_ref[0] + degp_ref[1] + 1.0
    dinv = lax.rsqrt(deg)[:N]
    dinv_ref[...] = dinv
    h = jnp.dot(x_ref[...], w_ref[...], preferred_element_type=jnp.float32)
    g_ref[...] = dinv * h


@jax.jit
def _tc1(degp, x, w1):
    return pl.pallas_call(
        _tc1_body,
        out_shape=(
            jax.ShapeDtypeStruct((N, 1), jnp.float32),
            jax.ShapeDtypeStruct((N, D), jnp.float32),
        ),
    )(degp, x, w1)


def _tc2_body(p_ref, g_ref, dinv_ref, b_ref, w_ref, g2_ref):
    ssum = (p_ref[0] + p_ref[1])[:N] + g_ref[...]
    h = jnp.maximum(dinv_ref[...] * ssum + b_ref[...], 0.0)
    hw = jnp.dot(h, w_ref[...], preferred_element_type=jnp.float32)
    g2_ref[...] = dinv_ref[...] * hw


@jax.jit
def _tc2(p, g, dinv, b, w2):
    return pl.pallas_call(
        _tc2_body,
        out_shape=jax.ShapeDtypeStruct((N, D), jnp.float32),
    )(p, g, dinv, b, w2)


def _tc3_body(p_ref, g_ref, dinv_ref, b_ref, wr1_ref, br1_ref, wr2_ref,
              br2_ref, o_ref):
    ssum = (p_ref[0] + p_ref[1])[:N] + g_ref[...]
    h = jnp.maximum(dinv_ref[...] * ssum + b_ref[...], 0.0)
    t = jnp.maximum(
        jnp.dot(h, wr1_ref[...], preferred_element_type=jnp.float32)
        + br1_ref[...], 0.0)
    o_ref[...] = (
        jnp.dot(t, wr2_ref[...], preferred_element_type=jnp.float32)
        + br2_ref[...])


@jax.jit
def _tc3(p, g, dinv, b, wr1, br1, wr2, br2):
    return pl.pallas_call(
        _tc3_body,
        out_shape=jax.ShapeDtypeStruct((N, D_OUT), jnp.float32),
    )(p, g, dinv, b, wr1, br1, wr2, br2)


@jax.jit
def kernel(x, edge_index, W1, b1, W2, b2, Wr1, br1, Wr2, br2):
    src = edge_index[0]
    dst = edge_index[1]
    zeros_nd = jnp.zeros((NP, D), jnp.float32)

    degp = _sc_deg(dst).reshape(NC, NP, 1)
    dinv, g1 = _tc1(degp, x, W1)
    p1 = _sc_agg(g1, src, dst, zeros_nd)
    g2 = _tc2(p1, g1, dinv, b1.reshape(1, D), W2)
    p2 = _sc_agg(g2, src, dst, zeros_nd)
    out = _tc3(p2, g2, dinv, b2.reshape(1, D), Wr1, br1.reshape(1, -1),
               Wr2, br2.reshape(1, -1))
    return out


# speedup vs baseline: 32.5921x; 1.0013x over previous
import jax
import jax.numpy as jnp
from jax import lax
from jax.experimental import pallas as pl
from jax.experimental.pallas import tpu as pltpu
from jax.experimental.pallas import tpu_sc as plsc

N = 10000
E = 320000
D = 128
D_OUT = 64

NC = 2
NS = 16
NW = NC * NS

EPW = E // NW
C = 128
NB = 78
CT = EPW - NB * C

NP = 10240
RPT = NP // NS
RPT_D = RPT


HR = NP // 128


def _sc_deg_body(dst_hbm, out_hbm, didx_all, rowidx, zrows, hist, acc):
    c = lax.axis_index("c")
    s = lax.axis_index("s")
    wid = c * NS + s
    ebase = wid * EPW
    zero16 = jnp.zeros((16,), jnp.float32)
    one16 = jnp.ones((16,), jnp.float32)

    def zhist(i, carry):
        hist[i // 8, pl.ds((i % 8) * 16, 16)] = zero16
        return carry

    lax.fori_loop(0, HR * 8, zhist, 0)

    for j in range(HR // 16):
        rowidx[pl.ds(j * 16, 16)] = lax.iota(jnp.int32, 16) + j * 16

    @pl.when(s < 5)
    def _():
        def zr(i, carry):
            zrows[i // 8, pl.ds((i % 8) * 16, 16)] = zero16
            return carry
        lax.fori_loop(0, 16 * 8, zr, 0)
        pltpu.sync_copy(zrows, acc.at[pl.ds(s * 16, 16)])
    plsc.subcore_barrier()

    pltpu.sync_copy(dst_hbm.at[pl.ds(ebase, EPW)], didx_all)

    def step(j, carry):
        for k in range(4):
            idx = didx_all[pl.ds(j * 64 + k * 16, 16)]
            r = jax.lax.shift_right_logical(idx, 7)
            l = jax.lax.bitwise_and(idx, 127)
            plsc.addupdate_scatter(hist, [r, l], one16)
        return carry

    lax.fori_loop(0, EPW // 64, step, 0)

    for t in range((EPW % 64) // 16):
        idx = didx_all[pl.ds((EPW // 64) * 64 + t * 16, 16)]
        r = jax.lax.shift_right_logical(idx, 7)
        l = jax.lax.bitwise_and(idx, 127)
        plsc.addupdate_scatter(hist, [r, l], one16)

    pltpu.sync_copy(hist, acc.at[rowidx], add=True)
    plsc.subcore_barrier()

    @pl.when(s < 5)
    def _():
        pltpu.sync_copy(acc.at[pl.ds(s * 16, 16)],
                        out_hbm.at[c, pl.ds(s * 16, 16)])


@jax.jit
def _sc_deg(dst):
    return pl.kernel(
        _sc_deg_body,
        out_type=jax.ShapeDtypeStruct((NC, HR, 128), jnp.float32),
        mesh=plsc.VectorSubcoreMesh(core_axis_name="c", subcore_axis_name="s"),
        scratch_types=[
            pltpu.VMEM((EPW,), jnp.int32),
            pltpu.VMEM((HR,), jnp.int32),
            pltpu.VMEM((16, 128), jnp.float32),
            pltpu.VMEM((HR, 128), jnp.float32),
            pltpu.VMEM_SHARED((HR, 128), jnp.float32),
        ],
        compiler_params=pltpu.CompilerParams(needs_layout_passes=False),
    )(dst)


def _sc_agg_body(g_hbm, src_hbm, dst_hbm, zeros_hbm, out_hbm,
                 sidx_all, didx0, didx1, tidx, rows0, rows1, rowst,
                 acc, gsem0, gsem1, isem0, isem1):
    c = lax.axis_index("c")
    s = lax.axis_index("s")
    wid = c * NS + s
    ebase = wid * EPW

    pltpu.sync_copy(zeros_hbm.at[pl.ds(s * RPT, RPT)],
                    acc.at[pl.ds(s * RPT, RPT)])
    pltpu.sync_copy(src_hbm.at[pl.ds(ebase, EPW)], sidx_all)
    plsc.subcore_barrier()

    def gather_start(j, rows, gsem):
        pltpu.async_copy(g_hbm.at[sidx_all.at[pl.ds(j * C, C)]], rows, gsem)

    def gather_wait(j, rows, gsem):
        pltpu.make_async_copy(g_hbm.at[sidx_all.at[pl.ds(j * C, C)]],
                              rows, gsem).wait()

    def didx_start(j, didx, isem):
        pltpu.async_copy(dst_hbm.at[pl.ds(ebase + j * C, C)], didx, isem)

    def didx_wait(j, didx, isem):
        pltpu.make_async_copy(dst_hbm.at[pl.ds(ebase + j * C, C)],
                              didx, isem).wait()

    didx_start(0, didx0, isem0)
    gather_start(0, rows0, gsem0)

    def pair(i, carry):
        j0 = 2 * i
        didx_start(j0 + 1, didx1, isem1)
        gather_start(j0 + 1, rows1, gsem1)
        gather_wait(j0, rows0, gsem0)
        didx_wait(j0, didx0, isem0)
        pltpu.sync_copy(rows0, acc.at[didx0], add=True)
        didx_start(j0 + 2, didx0, isem0)
        gather_start(j0 + 2, rows0, gsem0)
        gather_wait(j0 + 1, rows1, gsem1)
        didx_wait(j0 + 1, didx1, isem1)
        pltpu.sync_copy(rows1, acc.at[didx1], add=True)
        return carry

    lax.fori_loop(0, NB // 2 - 1, pair, 0)

    didx_start(NB - 1, didx1, isem1)
    gather_start(NB - 1, rows1, gsem1)
    gather_wait(NB - 2, rows0, gsem0)
    didx_wait(NB - 2, didx0, isem0)
    pltpu.sync_copy(rows0, acc.at[didx0], add=True)
    pltpu.sync_copy(dst_hbm.at[pl.ds(ebase + NB * C, CT)], tidx)
    pltpu.async_copy(g_hbm.at[sidx_all.at[pl.ds(NB * C, CT)]], rowst, gsem0)
    gather_wait(NB - 1, rows1, gsem1)
    didx_wait(NB - 1, didx1, isem1)
    pltpu.sync_copy(rows1, acc.at[didx1], add=True)
    pltpu.make_async_copy(g_hbm.at[sidx_all.at[pl.ds(NB * C, CT)]],
                          rowst, gsem0).wait()
    pltpu.sync_copy(rowst, acc.at[tidx], add=True)
    plsc.subcore_barrier()

    pltpu.sync_copy(acc.at[pl.ds(s * RPT, RPT)],
                    out_hbm.at[c, pl.ds(s * RPT, RPT)])


@jax.jit
def _sc_agg(g, src, dst, zeros_nd):
    return pl.kernel(
        _sc_agg_body,
        out_type=jax.ShapeDtypeStruct((NC, NP, D), jnp.float32),
        mesh=plsc.VectorSubcoreMesh(core_axis_name="c", subcore_axis_name="s"),
        scratch_types=[
            pltpu.VMEM((EPW,), jnp.int32),
            pltpu.VMEM((C,), jnp.int32),
            pltpu.VMEM((C,), jnp.int32),
            pltpu.VMEM((CT,), jnp.int32),
            pltpu.VMEM((C, D), jnp.float32),
            pltpu.VMEM((C, D), jnp.float32),
            pltpu.VMEM((CT, D), jnp.float32),
            pltpu.VMEM_SHARED((NP, D), jnp.float32),
            pltpu.SemaphoreType.DMA,
            pltpu.SemaphoreType.DMA,
            pltpu.SemaphoreType.DMA,
            pltpu.SemaphoreType.DMA,
        ],
    )(g, src, dst, zeros_nd)


def _tc1_body(degp_ref, x_ref, w_ref, dinv_ref, g_ref):
    deg = degp_ref[0] + degp_ref[1] + 1.0
    dinv = lax.rsqrt(deg)[:N]
    dinv_ref[...] = dinv
    h = jnp.dot(x_ref[...], w_ref[...], preferred_element_type=jnp.float32)
    g_ref[...] = dinv * h


@jax.jit
def _tc1(degp, x, w1):
    return pl.pallas_call(
        _tc1_body,
        out_shape=(
            jax.ShapeDtypeStruct((N, 1), jnp.float32),
            jax.ShapeDtypeStruct((N, D), jnp.float32),
        ),
    )(degp, x, w1)


def _tc2_body(p_ref, g_ref, dinv_ref, b_ref, w_ref, g2_ref):
    ssum = (p_ref[0] + p_ref[1])[:N] + g_ref[...]
    h = jnp.maximum(dinv_ref[...] * ssum + b_ref[...], 0.0)
    hw = jnp.dot(h, w_ref[...], preferred_element_type=jnp.float32)
    g2_ref[...] = dinv_ref[...] * hw


@jax.jit
def _tc2(p, g, dinv, b, w2):
    return pl.pallas_call(
        _tc2_body,
        out_shape=jax.ShapeDtypeStruct((N, D), jnp.float32),
    )(p, g, dinv, b, w2)


def _tc3_body(p_ref, g_ref, dinv_ref, b_ref, wr1_ref, br1_ref, wr2_ref,
              br2_ref, o_ref):
    ssum = (p_ref[0] + p_ref[1])[:N] + g_ref[...]
    h = jnp.maximum(dinv_ref[...] * ssum + b_ref[...], 0.0)
    t = jnp.maximum(
        jnp.dot(h, wr1_ref[...], preferred_element_type=jnp.float32)
        + br1_ref[...], 0.0)
    o_ref[...] = (
        jnp.dot(t, wr2_ref[...], preferred_element_type=jnp.float32)
        + br2_ref[...])


@jax.jit
def _tc3(p, g, dinv, b, wr1, br1, wr2, br2):
    return pl.pallas_call(
        _tc3_body,
        out_shape=jax.ShapeDtypeStruct((N, D_OUT), jnp.float32),
    )(p, g, dinv, b, wr1, br1, wr2, br2)


@jax.jit
def kernel(x, edge_index, W1, b1, W2, b2, Wr1, br1, Wr2, br2):
    src = edge_index[0]
    dst = edge_index[1]
    zeros_nd = jnp.zeros((NP, D), jnp.float32)

    degp = _sc_deg(dst).reshape(NC, NP, 1)
    dinv, g1 = _tc1(degp, x, W1)
    p1 = _sc_agg(g1, src, dst, zeros_nd)
    g2 = _tc2(p1, g1, dinv, b1.reshape(1, D), W2)
    p2 = _sc_agg(g2, src, dst, zeros_nd)
    out = _tc3(p2, g2, dinv, b2.reshape(1, D), Wr1, br1.reshape(1, -1),
               Wr2, br2.reshape(1, -1))
    return out


# speedup vs baseline: 35.0059x; 1.0741x over previous
import jax
import jax.numpy as jnp
from jax import lax
from jax.experimental import pallas as pl
from jax.experimental.pallas import tpu as pltpu
from jax.experimental.pallas import tpu_sc as plsc

N = 10000
E = 320000
D = 128
D_OUT = 64

NC = 2
NS = 16
NW = NC * NS

EPW = E // NW
C = 80
NCHUNK = EPW // C

NP = 10240
RPT = NP // NS
RPT_D = RPT


HR = NP // 128


def _sc_deg_body(dst_hbm, out_hbm, didx_all, rowidx, zrows, hist, acc):
    c = lax.axis_index("c")
    s = lax.axis_index("s")
    wid = c * NS + s
    ebase = wid * EPW
    zero16 = jnp.zeros((16,), jnp.float32)
    one16 = jnp.ones((16,), jnp.float32)

    def zhist(i, carry):
        hist[i // 8, pl.ds((i % 8) * 16, 16)] = zero16
        return carry

    lax.fori_loop(0, HR * 8, zhist, 0)

    for j in range(HR // 16):
        rowidx[pl.ds(j * 16, 16)] = lax.iota(jnp.int32, 16) + j * 16

    @pl.when(s < 5)
    def _():
        def zr(i, carry):
            zrows[i // 8, pl.ds((i % 8) * 16, 16)] = zero16
            return carry
        lax.fori_loop(0, 16 * 8, zr, 0)
        pltpu.sync_copy(zrows, acc.at[pl.ds(s * 16, 16)])
    plsc.subcore_barrier()

    pltpu.sync_copy(dst_hbm.at[pl.ds(ebase, EPW)], didx_all)

    def step(j, carry):
        for k in range(4):
            idx = didx_all[pl.ds(j * 64 + k * 16, 16)]
            r = jax.lax.shift_right_logical(idx, 7)
            l = jax.lax.bitwise_and(idx, 127)
            plsc.addupdate_scatter(hist, [r, l], one16)
        return carry

    lax.fori_loop(0, EPW // 64, step, 0)

    for t in range((EPW % 64) // 16):
        idx = didx_all[pl.ds((EPW // 64) * 64 + t * 16, 16)]
        r = jax.lax.shift_right_logical(idx, 7)
        l = jax.lax.bitwise_and(idx, 127)
        plsc.addupdate_scatter(hist, [r, l], one16)

    pltpu.sync_copy(hist, acc.at[rowidx], add=True)
    plsc.subcore_barrier()

    @pl.when(s < 5)
    def _():
        pltpu.sync_copy(acc.at[pl.ds(s * 16, 16)],
                        out_hbm.at[c, pl.ds(s * 16, 16)])


@jax.jit
def _sc_deg(dst):
    return pl.kernel(
        _sc_deg_body,
        out_type=jax.ShapeDtypeStruct((NC, HR, 128), jnp.float32),
        mesh=plsc.VectorSubcoreMesh(core_axis_name="c", subcore_axis_name="s"),
        scratch_types=[
            pltpu.VMEM((EPW,), jnp.int32),
            pltpu.VMEM((HR,), jnp.int32),
            pltpu.VMEM((16, 128), jnp.float32),
            pltpu.VMEM((HR, 128), jnp.float32),
            pltpu.VMEM_SHARED((HR, 128), jnp.float32),
        ],
        compiler_params=pltpu.CompilerParams(needs_layout_passes=False),
    )(dst)


def _sc_agg_body(g_hbm, src_hbm, dst_hbm, zeros_hbm, out_hbm,
                 sidx_all, didx0, didx1, didx2, rows0, rows1, rows2,
                 acc, gsem0, gsem1, gsem2, isem0, isem1, isem2):
    c = lax.axis_index("c")
    s = lax.axis_index("s")
    wid = c * NS + s
    ebase = wid * EPW

    didx = (didx0, didx1, didx2)
    rows = (rows0, rows1, rows2)
    gsem = (gsem0, gsem1, gsem2)
    isem = (isem0, isem1, isem2)

    pltpu.sync_copy(zeros_hbm.at[pl.ds(s * RPT, RPT)],
                    acc.at[pl.ds(s * RPT, RPT)])
    pltpu.sync_copy(src_hbm.at[pl.ds(ebase, EPW)], sidx_all)
    plsc.subcore_barrier()

    def start(j, k):
        pltpu.async_copy(dst_hbm.at[pl.ds(ebase + j * C, C)], didx[k],
                         isem[k])
        pltpu.async_copy(g_hbm.at[sidx_all.at[pl.ds(j * C, C)]], rows[k],
                         gsem[k])

    def finish(j, k):
        pltpu.make_async_copy(g_hbm.at[sidx_all.at[pl.ds(j * C, C)]],
                              rows[k], gsem[k]).wait()
        pltpu.make_async_copy(dst_hbm.at[pl.ds(ebase + j * C, C)], didx[k],
                              isem[k]).wait()
        pltpu.sync_copy(rows[k], acc.at[didx[k]], add=True)

    start(0, 0)
    start(1, 1)

    def triple(i, carry):
        j = 3 * i
        for k in range(3):
            start(j + k + 2, (k + 2) % 3)
            finish(j + k, k)
        return carry

    lax.fori_loop(0, (NCHUNK - 2) // 3, triple, 0)
    finish(NCHUNK - 2, (NCHUNK - 2) % 3)
    finish(NCHUNK - 1, (NCHUNK - 1) % 3)
    plsc.subcore_barrier()

    pltpu.sync_copy(acc.at[pl.ds(s * RPT, RPT)],
                    out_hbm.at[c, pl.ds(s * RPT, RPT)])


@jax.jit
def _sc_agg(g, src, dst, zeros_nd):
    return pl.kernel(
        _sc_agg_body,
        out_type=jax.ShapeDtypeStruct((NC, NP, D), jnp.float32),
        mesh=plsc.VectorSubcoreMesh(core_axis_name="c", subcore_axis_name="s"),
        scratch_types=[
            pltpu.VMEM((EPW,), jnp.int32),
            pltpu.VMEM((C,), jnp.int32),
            pltpu.VMEM((C,), jnp.int32),
            pltpu.VMEM((C,), jnp.int32),
            pltpu.VMEM((C, D), jnp.float32),
            pltpu.VMEM((C, D), jnp.float32),
            pltpu.VMEM((C, D), jnp.float32),
            pltpu.VMEM_SHARED((NP, D), jnp.float32),
            pltpu.SemaphoreType.DMA,
            pltpu.SemaphoreType.DMA,
            pltpu.SemaphoreType.DMA,
            pltpu.SemaphoreType.DMA,
            pltpu.SemaphoreType.DMA,
            pltpu.SemaphoreType.DMA,
        ],
    )(g, src, dst, zeros_nd)


def _tc1_body(degp_ref, x_ref, w_ref, dinv_ref, g_ref):
    deg = degp_ref[0] + degp_ref[1] + 1.0
    dinv = lax.rsqrt(deg)[:N]
    dinv_ref[...] = dinv
    h = jnp.dot(x_ref[...], w_ref[...], preferred_element_type=jnp.float32)
    g_ref[...] = dinv * h


@jax.jit
def _tc1(degp, x, w1):
    return pl.pallas_call(
        _tc1_body,
        out_shape=(
            jax.ShapeDtypeStruct((N, 1), jnp.float32),
            jax.ShapeDtypeStruct((N, D), jnp.float32),
        ),
    )(degp, x, w1)


def _tc2_body(p_ref, g_ref, dinv_ref, b_ref, w_ref, g2_ref):
    ssum = (p_ref[0] + p_ref[1])[:N] + g_ref[...]
    h = jnp.maximum(dinv_ref[...] * ssum + b_ref[...], 0.0)
    hw = jnp.dot(h, w_ref[...], preferred_element_type=jnp.float32)
    g2_ref[...] = dinv_ref[...] * hw


@jax.jit
def _tc2(p, g, dinv, b, w2):
    return pl.pallas_call(
        _tc2_body,
        out_shape=jax.ShapeDtypeStruct((N, D), jnp.float32),
    )(p, g, dinv, b, w2)


def _tc3_body(p_ref, g_ref, dinv_ref, b_ref, wr1_ref, br1_ref, wr2_ref,
              br2_ref, o_ref):
    ssum = (p_ref[0] + p_ref[1])[:N] + g_ref[...]
    h = jnp.maximum(dinv_ref[...] * ssum + b_ref[...], 0.0)
    t = jnp.maximum(
        jnp.dot(h, wr1_ref[...], preferred_element_type=jnp.float32)
        + br1_ref[...], 0.0)
    o_ref[...] = (
        jnp.dot(t, wr2_ref[...], preferred_element_type=jnp.float32)
        + br2_ref[...])


@jax.jit
def _tc3(p, g, dinv, b, wr1, br1, wr2, br2):
    return pl.pallas_call(
        _tc3_body,
        out_shape=jax.ShapeDtypeStruct((N, D_OUT), jnp.float32),
    )(p, g, dinv, b, wr1, br1, wr2, br2)


@jax.jit
def kernel(x, edge_index, W1, b1, W2, b2, Wr1, br1, Wr2, br2):
    src = edge_index[0]
    dst = edge_index[1]
    zeros_nd = jnp.zeros((NP, D), jnp.float32)

    degp = _sc_deg(dst).reshape(NC, NP, 1)
    dinv, g1 = _tc1(degp, x, W1)
    p1 = _sc_agg(g1, src, dst, zeros_nd)
    g2 = _tc2(p1, g1, dinv, b1.reshape(1, D), W2)
    p2 = _sc_agg(g2, src, dst, zeros_nd)
    out = _tc3(p2, g2, dinv, b2.reshape(1, D), Wr1, br1.reshape(1, -1),
               Wr2, br2.reshape(1, -1))
    return out


# speedup vs baseline: 37.6904x; 1.0767x over previous
import jax
import jax.numpy as jnp
from jax import lax
from jax.experimental import pallas as pl
from jax.experimental.pallas import tpu as pltpu
from jax.experimental.pallas import tpu_sc as plsc

N = 10000
E = 320000
D = 128
D_OUT = 64

NC = 2
NS = 16
NW = NC * NS

EPW = E // NW
C = 80
NCHUNK = EPW // C

NP = 10240
RPT = NP // NS
RPT_D = RPT


HR = NP // 128


def _sc_deg_body(ei_hbm, out_hbm, didx_all, rowidx, zrows, hist, acc):
    c = lax.axis_index("c")
    s = lax.axis_index("s")
    wid = c * NS + s
    ebase = wid * EPW
    zero16 = jnp.zeros((16,), jnp.float32)
    one16 = jnp.ones((16,), jnp.float32)

    def zhist(i, carry):
        hist[i // 8, pl.ds((i % 8) * 16, 16)] = zero16
        return carry

    lax.fori_loop(0, HR * 8, zhist, 0)

    for j in range(HR // 16):
        rowidx[pl.ds(j * 16, 16)] = lax.iota(jnp.int32, 16) + j * 16

    @pl.when(s < 5)
    def _():
        def zr(i, carry):
            zrows[i // 8, pl.ds((i % 8) * 16, 16)] = zero16
            return carry
        lax.fori_loop(0, 16 * 8, zr, 0)
        pltpu.sync_copy(zrows, acc.at[pl.ds(s * 16, 16)])
    plsc.subcore_barrier()

    pltpu.sync_copy(ei_hbm.at[pl.ds(E + ebase, EPW)], didx_all)

    def step(j, carry):
        for k in range(4):
            idx = didx_all[pl.ds(j * 64 + k * 16, 16)]
            r = jax.lax.shift_right_logical(idx, 7)
            l = jax.lax.bitwise_and(idx, 127)
            plsc.addupdate_scatter(hist, [r, l], one16)
        return carry

    lax.fori_loop(0, EPW // 64, step, 0)

    for t in range((EPW % 64) // 16):
        idx = didx_all[pl.ds((EPW // 64) * 64 + t * 16, 16)]
        r = jax.lax.shift_right_logical(idx, 7)
        l = jax.lax.bitwise_and(idx, 127)
        plsc.addupdate_scatter(hist, [r, l], one16)

    pltpu.sync_copy(hist, acc.at[rowidx], add=True)
    plsc.subcore_barrier()

    @pl.when(s < 5)
    def _():
        pltpu.sync_copy(acc.at[pl.ds(s * 16, 16)],
                        out_hbm.at[c, pl.ds(s * 16, 16)])


@jax.jit
def _sc_deg(ei_flat):
    return pl.kernel(
        _sc_deg_body,
        out_type=jax.ShapeDtypeStruct((NC, HR, 128), jnp.float32),
        mesh=plsc.VectorSubcoreMesh(core_axis_name="c", subcore_axis_name="s"),
        scratch_types=[
            pltpu.VMEM((EPW,), jnp.int32),
            pltpu.VMEM((HR,), jnp.int32),
            pltpu.VMEM((16, 128), jnp.float32),
            pltpu.VMEM((HR, 128), jnp.float32),
            pltpu.VMEM_SHARED((HR, 128), jnp.float32),
        ],
        compiler_params=pltpu.CompilerParams(needs_layout_passes=False),
    )(ei_flat)


def _sc_agg_body(g_hbm, ei_hbm, zeros_hbm, out_hbm,
                 sidx_all, didx0, didx1, didx2, rows0, rows1, rows2,
                 acc, gsem0, gsem1, gsem2, isem0, isem1, isem2):
    c = lax.axis_index("c")
    s = lax.axis_index("s")
    wid = c * NS + s
    ebase = wid * EPW

    didx = (didx0, didx1, didx2)
    rows = (rows0, rows1, rows2)
    gsem = (gsem0, gsem1, gsem2)
    isem = (isem0, isem1, isem2)

    pltpu.sync_copy(zeros_hbm.at[pl.ds(s * RPT, RPT)],
                    acc.at[pl.ds(s * RPT, RPT)])
    pltpu.sync_copy(ei_hbm.at[pl.ds(ebase, EPW)], sidx_all)
    plsc.subcore_barrier()

    def start(j, k):
        pltpu.async_copy(ei_hbm.at[pl.ds(E + ebase + j * C, C)], didx[k],
                         isem[k])
        pltpu.async_copy(g_hbm.at[sidx_all.at[pl.ds(j * C, C)]], rows[k],
                         gsem[k])

    def finish(j, k):
        pltpu.make_async_copy(g_hbm.at[sidx_all.at[pl.ds(j * C, C)]],
                              rows[k], gsem[k]).wait()
        pltpu.make_async_copy(ei_hbm.at[pl.ds(E + ebase + j * C, C)], didx[k],
                              isem[k]).wait()
        pltpu.sync_copy(rows[k], acc.at[didx[k]], add=True)

    start(0, 0)
    start(1, 1)

    def triple(i, carry):
        j = 3 * i
        for k in range(3):
            start(j + k + 2, (k + 2) % 3)
            finish(j + k, k)
        return carry

    lax.fori_loop(0, (NCHUNK - 2) // 3, triple, 0)
    finish(NCHUNK - 2, (NCHUNK - 2) % 3)
    finish(NCHUNK - 1, (NCHUNK - 1) % 3)
    plsc.subcore_barrier()

    pltpu.sync_copy(acc.at[pl.ds(s * RPT, RPT)],
                    out_hbm.at[c, pl.ds(s * RPT, RPT)])


@jax.jit
def _sc_agg(g, ei_flat, zeros_nd):
    return pl.kernel(
        _sc_agg_body,
        out_type=jax.ShapeDtypeStruct((NC, NP, D), jnp.float32),
        mesh=plsc.VectorSubcoreMesh(core_axis_name="c", subcore_axis_name="s"),
        scratch_types=[
            pltpu.VMEM((EPW,), jnp.int32),
            pltpu.VMEM((C,), jnp.int32),
            pltpu.VMEM((C,), jnp.int32),
            pltpu.VMEM((C,), jnp.int32),
            pltpu.VMEM((C, D), jnp.float32),
            pltpu.VMEM((C, D), jnp.float32),
            pltpu.VMEM((C, D), jnp.float32),
            pltpu.VMEM_SHARED((NP, D), jnp.float32),
            pltpu.SemaphoreType.DMA,
            pltpu.SemaphoreType.DMA,
            pltpu.SemaphoreType.DMA,
            pltpu.SemaphoreType.DMA,
            pltpu.SemaphoreType.DMA,
            pltpu.SemaphoreType.DMA,
        ],
    )(g, ei_flat, zeros_nd)


def _tc0_body(degp_ref, dinv2_ref):
    dinv2_ref[...] = lax.rsqrt(degp_ref[0] + degp_ref[1] + 1.0)


@jax.jit
def _tc0(degp):
    return pl.pallas_call(
        _tc0_body,
        out_shape=jax.ShapeDtypeStruct((HR, 128), jnp.float32),
    )(degp)


def _tc1_body(dinv_ref, x_ref, w_ref, g_ref):
    h = jnp.dot(x_ref[...], w_ref[...], preferred_element_type=jnp.float32)
    g_ref[...] = dinv_ref[:N] * h


@jax.jit
def _tc1(dinv, x, w1):
    return pl.pallas_call(
        _tc1_body,
        out_shape=jax.ShapeDtypeStruct((N, D), jnp.float32),
    )(dinv, x, w1)


def _tc2_body(p_ref, g_ref, dinv_ref, b_ref, w_ref, g2_ref):
    dinv = dinv_ref[:N]
    ssum = (p_ref[0] + p_ref[1])[:N] + g_ref[...]
    h = jnp.maximum(dinv * ssum + b_ref[...], 0.0)
    hw = jnp.dot(h, w_ref[...], preferred_element_type=jnp.float32)
    g2_ref[...] = dinv * hw


@jax.jit
def _tc2(p, g, dinv, b, w2):
    return pl.pallas_call(
        _tc2_body,
        out_shape=jax.ShapeDtypeStruct((N, D), jnp.float32),
    )(p, g, dinv, b, w2)


def _tc3_body(p_ref, g_ref, dinv_ref, b_ref, wr1_ref, br1_ref, wr2_ref,
              br2_ref, o_ref):
    dinv = dinv_ref[:N]
    ssum = (p_ref[0] + p_ref[1])[:N] + g_ref[...]
    h = jnp.maximum(dinv * ssum + b_ref[...], 0.0)
    t = jnp.maximum(
        jnp.dot(h, wr1_ref[...], preferred_element_type=jnp.float32)
        + br1_ref[...], 0.0)
    o_ref[...] = (
        jnp.dot(t, wr2_ref[...], preferred_element_type=jnp.float32)
        + br2_ref[...])


@jax.jit
def _tc3(p, g, dinv, b, wr1, br1, wr2, br2):
    return pl.pallas_call(
        _tc3_body,
        out_shape=jax.ShapeDtypeStruct((N, D_OUT), jnp.float32),
    )(p, g, dinv, b, wr1, br1, wr2, br2)


@jax.jit
def kernel(x, edge_index, W1, b1, W2, b2, Wr1, br1, Wr2, br2):
    ei_flat = edge_index.reshape(2 * E)
    zeros_nd = jnp.zeros((NP, D), jnp.float32)

    degp = _sc_deg(ei_flat)
    dinv = _tc0(degp).reshape(NP, 1)
    g1 = _tc1(dinv, x, W1)
    p1 = _sc_agg(g1, ei_flat, zeros_nd)
    g2 = _tc2(p1, g1, dinv, b1.reshape(1, D), W2)
    p2 = _sc_agg(g2, ei_flat, zeros_nd)
    out = _tc3(p2, g2, dinv, b2.reshape(1, D), Wr1, br1.reshape(1, -1),
               Wr2, br2.reshape(1, -1))
    return out
